# Initial kernel scaffold; baseline (speedup 1.0000x reference)
import jax
import jax.numpy as jnp
from jax.experimental import pallas as pl


def kernel(x, ln_gamma, ln_beta, W, b):
    raise NotImplementedError("write your pallas kernel here")


# speedup vs baseline: 2.6170x; 2.6170x over previous
import functools

import jax
import jax.numpy as jnp
from jax import lax
from jax.experimental import pallas as pl
from jax.experimental.pallas import tpu as pltpu
from jax.experimental.pallas import tpu_sc as plsc

B, NS, D = 4, 8192, 1024
K = 512
DROP_KEEP = 0.9
SROW, SCOL = 64, 128

NC, NSUB = 2, 16
NW = NC * NSUB
ROWS_PER_W = (B * K) // NW
GRP = 16
NGRP = ROWS_PER_W // GRP


def _row_sum_seq(xb):
    acc = xb[:, 0:128]
    for c in range(1, 8):
        acc = acc + xb[:, c * 128:(c + 1) * 128]
    return jnp.sum(acc, axis=-1, keepdims=True)


def _row_sum_tree8(xb):
    ch = [xb[:, c * 128:(c + 1) * 128] for c in range(8)]
    s01, s23 = ch[0] + ch[1], ch[2] + ch[3]
    s45, s67 = ch[4] + ch[5], ch[6] + ch[7]
    return jnp.sum((s01 + s23) + (s45 + s67), axis=-1, keepdims=True)


def _score_body(x_ref, g_ref, be_ref, w_ref, b_ref, gum_ref, pen_ref, o_ref,
                v=0):
    xb = x_ref[0]
    if v >= 10:
        w = w_ref[...]
        if v == 10:
            z = jnp.dot(xb, w)[:, 0:1]
        elif v == 11:
            za = jnp.dot(xb[:, :512], w[:512])[:, 0:1]
            zb = jnp.dot(xb[:, 512:], w[512:])[:, 0:1]
            z = za + zb
        elif v == 12:
            parts = [jnp.dot(xb[:, c * 256:(c + 1) * 256],
                             w[c * 256:(c + 1) * 256])[:, 0:1]
                     for c in range(4)]
            z = ((parts[0] + parts[1]) + parts[2]) + parts[3]
        else:
            parts = [jnp.dot(xb[:, c * 256:(c + 1) * 256],
                             w[c * 256:(c + 1) * 256])[:, 0:1]
                     for c in range(4)]
            z = (parts[0] + parts[1]) + (parts[2] + parts[3])
        o_ref[0] = z
        return
    if v == 0:
        mu = jnp.mean(xb, axis=-1, keepdims=True)
        d = xb - mu
        var = jnp.mean(d * d, axis=-1, keepdims=True)
    elif v == 1:
        mu = jnp.sum(xb, axis=-1, keepdims=True) * (1.0 / D)
        d = xb - mu
        var = jnp.sum(d * d, axis=-1, keepdims=True) * (1.0 / D)
    elif v == 2:
        mu = _row_sum_seq(xb) * (1.0 / D)
        d = xb - mu
        var = _row_sum_seq(d * d) * (1.0 / D)
    else:
        mu = _row_sum_tree8(xb) * (1.0 / D)
        d = xb - mu
        var = _row_sum_tree8(d * d) * (1.0 / D)
    xn = d / jnp.sqrt(var + 1e-5) * g_ref[...] + be_ref[...]
    z = jnp.dot(xn, w_ref[...])[:, 0:1]
    s = (z + b_ref[0, 0]) / jnp.sqrt(2.0)
    s = s + gum_ref[0]
    s = s + pen_ref[0]
    o_ref[0] = s


def _scores(x, ln_gamma, ln_beta, W, b, gum, pen, variant=0):
    CH = 512
    grid = (B, NS // CH)
    f = pl.pallas_call(
        functools.partial(_score_body, v=variant),
        grid=grid,
        in_specs=[
            pl.BlockSpec((1, CH, D), lambda i, j: (i, j, 0)),
            pl.BlockSpec((1, D), lambda i, j: (0, 0)),
            pl.BlockSpec((1, D), lambda i, j: (0, 0)),
            pl.BlockSpec((D, 128), lambda i, j: (0, 0)),
            pl.BlockSpec((1, 1), lambda i, j: (0, 0)),
            pl.BlockSpec((1, CH, 1), lambda i, j: (i, j, 0)),
            pl.BlockSpec((1, CH, 1), lambda i, j: (i, j, 0)),
        ],
        out_specs=pl.BlockSpec((1, CH, 1), lambda i, j: (i, j, 0)),
        out_shape=jax.ShapeDtypeStruct((B, NS, 1), jnp.float32),
    )
    wpad = jnp.pad(W, ((0, 0), (0, 127)))
    return f(x, ln_gamma.reshape(1, D), ln_beta.reshape(1, D), wpad,
             b.reshape(1, 1), gum, pen)


def _sort_body(s_ref, idx_ref, sp_ref):
    s = s_ref[0]
    row = lax.broadcasted_iota(jnp.int32, (SROW, SCOL), 0)
    col = lax.broadcasted_iota(jnp.int32, (SROW, SCOL), 1)
    pos = row * SCOL + col
    idx = pos
    kk = 2
    while kk <= NS:
        j = kk // 2
        while j >= 1:
            upper = (pos & j) != 0
            want_max = ((pos & kk) == 0) ^ upper
            if j < SCOL:
                ax, sh, n = 1, j, SCOL
            else:
                ax, sh, n = 0, j // SCOL, SROW
            ps = jnp.where(upper, pltpu.roll(s, sh, ax),
                           pltpu.roll(s, n - sh, ax))
            pidx = jnp.where(upper, pltpu.roll(idx, sh, ax),
                             pltpu.roll(idx, n - sh, ax))
            self_wins = (s > ps) | ((s == ps) & (idx < pidx))
            take_self = self_wins == want_max
            s = jnp.where(take_self, s, ps)
            idx = jnp.where(take_self, idx, pidx)
            j //= 2
        kk *= 2
    idx_ref[0] = idx
    sp_ref[0] = jax.nn.softplus(s)


def _sort(scores2):
    return pl.pallas_call(
        _sort_body,
        grid=(B,),
        in_specs=[pl.BlockSpec((1, SROW, SCOL), lambda i: (i, 0, 0))],
        out_specs=[pl.BlockSpec((1, SROW, SCOL), lambda i: (i, 0, 0)),
                   pl.BlockSpec((1, SROW, SCOL), lambda i: (i, 0, 0))],
        out_shape=[jax.ShapeDtypeStruct((B, SROW, SCOL), jnp.int32),
                   jax.ShapeDtypeStruct((B, SROW, SCOL), jnp.float32)],
    )(scores2)


def _sc_body(xflat, sidx, ssp, botrank, toprank, out_hbm, reg_hbm,
             sidx_v, ssp_v, br_v, w1a_v, w2a_v, xt_v, xb_v, out_v, reg_v,
             sem1, sem2):
    cid = lax.axis_index("c")
    sid = lax.axis_index("s")
    wid = sid * NC + cid
    g0 = wid * ROWS_PER_W
    bb = g0 // K
    j0 = g0 - bb * K

    pltpu.sync_copy(sidx.at[bb], sidx_v)
    pltpu.sync_copy(ssp.at[bb], ssp_v)
    pltpu.sync_copy(botrank.at[bb, pl.ds(j0, ROWS_PER_W)], br_v)

    regacc = jnp.zeros((16,), jnp.float32)
    for grp in range(NGRP):
        jvec = j0 + grp * GRP + lax.iota(jnp.int32, 16)
        bvec = br_v[pl.ds(grp * GRP, 16)]
        sp1 = plsc.load_gather(ssp_v, [jvec])
        sp2 = plsc.load_gather(ssp_v, [bvec])
        xs = sp1 + sp2 + 1e-9
        w1 = sp1 / xs
        w2 = sp2 / xs
        regacc = regacc + w2
        w1a_v[pl.ds(grp * GRP, 16)] = w1
        w2a_v[pl.ds(grp * GRP, 16)] = w2
    reg_v[...] = regacc
    for grp in range(NGRP):
        jvec = j0 + grp * GRP + lax.iota(jnp.int32, 16)
        bvec = br_v[pl.ds(grp * GRP, 16)]
        tidx = plsc.load_gather(sidx_v, [jvec])
        bidx = plsc.load_gather(sidx_v, [bvec])
        cp1 = pltpu.async_copy(xflat.at[tidx + bb * NS], xt_v, sem1)
        cp2 = pltpu.async_copy(xflat.at[bidx + bb * NS], xb_v, sem2)
        cp1.wait()
        cp2.wait()
        rw = []
        for r in range(GRP):
            sel = jnp.full((16,), grp * GRP + r, jnp.int32)
            rw.append((plsc.load_gather(w1a_v, [sel]),
                       plsc.load_gather(w2a_v, [sel])))

        def body(c, carry):
            sl = pl.ds(c * 16, 16)
            for r in range(GRP):
                out_v[r, sl] = xt_v[r, sl] * rw[r][0] + xb_v[r, sl] * rw[r][1]
            return carry

        lax.fori_loop(0, D // 16, body, 0)
        pltpu.sync_copy(out_v, out_hbm.at[pl.ds(g0 + grp * GRP, GRP)])
    pltpu.sync_copy(reg_v, reg_hbm.at[wid])


@functools.partial(jax.jit, static_argnums=())
def _sc_gather_combine(xflat, sidx, ssp, botrank, toprank):
    kfn = pl.kernel(
        _sc_body,
        out_type=[jax.ShapeDtypeStruct((B * K, D), jnp.float32),
                  jax.ShapeDtypeStruct((NW, 16), jnp.float32)],
        mesh=plsc.VectorSubcoreMesh(core_axis_name="c", subcore_axis_name="s",
                                    num_cores=NC, num_subcores=NSUB),
        compiler_params=pltpu.CompilerParams(needs_layout_passes=False),
        scratch_types=[
            pltpu.VMEM((NS,), jnp.int32),
            pltpu.VMEM((NS,), jnp.float32),
            pltpu.VMEM((ROWS_PER_W,), jnp.int32),
            pltpu.VMEM((ROWS_PER_W,), jnp.float32),
            pltpu.VMEM((ROWS_PER_W,), jnp.float32),
            pltpu.VMEM((GRP, D), jnp.float32),
            pltpu.VMEM((GRP, D), jnp.float32),
            pltpu.VMEM((GRP, D), jnp.float32),
            pltpu.VMEM((16,), jnp.float32),
            pltpu.SemaphoreType.DMA,
            pltpu.SemaphoreType.DMA,
        ],
    )
    return kfn(xflat, sidx, ssp, botrank, toprank)


def _noise_consts():
    rk = jax.random.key(42)
    rk1, rk2, rk3 = jax.random.split(rk, 3)
    u = jax.random.uniform(rk1, (B, NS, 1), dtype=jnp.float32)
    gum = -1.0 * jnp.log(-1.0 * jnp.log(u + 1e-20) + 1e-20)
    drop = jax.random.bernoulli(rk2, DROP_KEEP, (B, NS)).astype(jnp.float32)
    pen = (1.0 - drop)[:, :, None] * -1000000000.0
    perm = jnp.argsort(jax.random.uniform(rk3, (B, NS - K)), axis=1)
    botrank = (K + perm[:, :K]).astype(jnp.int32)
    return gum, pen, botrank


_DEBUG_JNP_TAIL = True
_DEBUG_SC_DIAG = False


def kernel(x, ln_gamma, ln_beta, W, b):
    assert x.shape == (B, NS, D)
    gum, pen, botrank = _noise_consts()
    scores = _scores(x, ln_gamma, ln_beta, W, b, gum, pen, variant=0)
    _DEBUG_JNP_SORT = False
    if _DEBUG_JNP_SORT:
        sidx = jnp.argsort(-scores[:, :, 0], axis=1).astype(jnp.int32)
        ssp = jax.nn.softplus(
            jnp.take_along_axis(scores[:, :, 0], sidx, axis=1))
    else:
        sidx, ssp = _sort(scores.reshape(B, SROW, SCOL))
        sidx = sidx.reshape(B, NS)
        ssp = ssp.reshape(B, NS)
    if _DEBUG_JNP_TAIL:
        ranks_top = jnp.arange(K)[None, :].repeat(B, 0)
        tidx = jnp.take_along_axis(sidx, ranks_top, 1)
        bidx = jnp.take_along_axis(sidx, botrank, 1)
        sp1 = jnp.take_along_axis(ssp, ranks_top, 1)
        sp2 = jnp.take_along_axis(ssp, botrank, 1)
        xs = sp1 + sp2 + 1e-9
        w1, w2 = sp1 / xs, sp2 / xs
        xt = jnp.take_along_axis(x, tidx[:, :, None], 1)
        xb = jnp.take_along_axis(x, bidx[:, :, None], 1)
        out = xt * w1[:, :, None] + xb * w2[:, :, None]
        reg = jnp.mean(w2)
        return (out, reg)
    toprank = jnp.tile(jnp.arange(K, dtype=jnp.int32)[None, :], (B, 1))
    outflat, regpart = _sc_gather_combine(
        x.reshape(B * NS, D), sidx, ssp, botrank, toprank)
    out = outflat.reshape(B, K, D)
    reg = jnp.sum(regpart) / (B * K)
    if _DEBUG_SC_DIAG:
        ranks_top = jnp.arange(K)[None, :].repeat(B, 0)
        tidx = jnp.take_along_axis(sidx, ranks_top, 1)
        bidx = jnp.take_along_axis(sidx, botrank, 1)
        sp1 = jnp.take_along_axis(ssp, ranks_top, 1)
        sp2 = jnp.take_along_axis(ssp, botrank, 1)
        xs = sp1 + sp2 + 1e-9
        w1, w2 = sp1 / xs, sp2 / xs
        xt = jnp.take_along_axis(x, tidx[:, :, None], 1)
        xbt = jnp.take_along_axis(x, bidx[:, :, None], 1)
        out_j = xt * w1[:, :, None] + xbt * w2[:, :, None]
        bad = jnp.max(jnp.abs(out - out_j), axis=-1) > 1e-3
        badf = bad.reshape(-1).astype(jnp.float32)
        nbad = jnp.minimum(jnp.sum(badf), 999.0)
        first = jnp.argmax(badf).astype(jnp.float32)
        enc = nbad * 10000.0 + first
        out_j = out_j.at[0, 0, 0].add(enc)
        return (out_j, jnp.mean(w2))
    return (out, reg)
